# baseline (device time: 80742 ns/iter reference)
import jax
import jax.numpy as jnp
from jax import lax
from jax.experimental import pallas as pl
from jax.experimental.pallas import tpu as pltpu

N_DEV = 4
N_HOP = N_DEV - 1
N_SUB = 2
_GELU_C = 0.7978845608028654


def _gelu(y):
    return 0.5 * y * (1.0 + jnp.tanh(_GELU_C * (y + 0.044715 * y * y * y)))


def kernel(x, w_mat):
    m_per, k = x.shape
    _, n_per = w_mat.shape
    m_half = m_per // 2
    m_sub = m_half // N_SUB

    def body(x_hbm, w_hbm, out_hbm,
             x_vmem, w_vmem, out_vmem, comm_a, comm_b,
             send_a, recv_a, send_b, recv_b, local_sems, out_sems):
        my = lax.axis_index("i")
        left = lax.rem(my - 1 + N_DEV, N_DEV)
        right = lax.rem(my + 1, N_DEV)

        cp_x = pltpu.make_async_copy(x_hbm, x_vmem, local_sems.at[0])
        cp_w = pltpu.make_async_copy(w_hbm, w_vmem, local_sems.at[1])
        cp_x.start()
        cp_w.start()

        barrier_sem = pltpu.get_barrier_semaphore()
        for nbr in (left, right):
            pl.semaphore_signal(
                barrier_sem, inc=1,
                device_id=(nbr,), device_id_type=pl.DeviceIdType.MESH,
            )
        pl.semaphore_wait(barrier_sem, 2)

        def make_rdma(ring_comm, sems_pair, dev, h, j):
            if h == 0:
                base = 0 if ring_comm is comm_a else m_half
                src = x_hbm.at[pl.ds(base + j * m_sub, m_sub)]
            else:
                src = ring_comm.at[h - 1, pl.ds(j * m_sub, m_sub)]
            send, recv = sems_pair
            return pltpu.make_async_remote_copy(
                src_ref=src,
                dst_ref=ring_comm.at[h, pl.ds(j * m_sub, m_sub)],
                send_sem=send.at[h, j],
                recv_sem=recv.at[h, j],
                device_id=(dev,),
                device_id_type=pl.DeviceIdType.MESH,
            )

        out_cps = []

        def store(row0, nrows, y, sem_idx):
            out_vmem[pl.ds(row0, nrows), :] = _gelu(y)
            cp = pltpu.make_async_copy(
                out_vmem.at[pl.ds(row0, nrows)],
                out_hbm.at[pl.ds(row0, nrows)],
                out_sems.at[sem_idx],
            )
            cp.start()
            out_cps.append(cp)

        rings = (
            (comm_a, (send_a, recv_a), right),
            (comm_b, (send_b, recv_b), left),
        )

        rdmas = {}
        for ring, sems, dev in rings:
            for j in range(N_SUB):
                r = make_rdma(ring, sems, dev, 0, j)
                r.start()
                rdmas[(id(ring), 0, j)] = r

        cp_w.wait()
        cp_x.wait()
        y = jnp.dot(x_vmem[:, :], w_vmem[:, :],
                    preferred_element_type=jnp.float32)
        store(my * m_per, m_per, y, 0)

        def compute(slot, sem_base):
            origin_a = lax.rem(my - slot - 1 + N_DEV, N_DEV)
            origin_b = lax.rem(my + slot + 1, N_DEV)
            ya = jnp.dot(comm_a[slot, :, :], w_vmem[:, :],
                         preferred_element_type=jnp.float32)
            store(origin_a * m_per, m_half, ya, sem_base)
            yb = jnp.dot(comm_b[slot, :, :], w_vmem[:, :],
                         preferred_element_type=jnp.float32)
            store(origin_b * m_per + m_half, m_half, yb, sem_base + 1)

        for h in range(1, N_HOP):
            for j in range(N_SUB):
                for ring, sems, dev in rings:
                    rdmas[(id(ring), h - 1, j)].wait_recv()
                    r = make_rdma(ring, sems, dev, h, j)
                    r.start()
                    rdmas[(id(ring), h, j)] = r
            compute(h - 1, 1 + 2 * (h - 1))

        s = N_HOP - 1
        origin_a = lax.rem(my - s - 1 + N_DEV, N_DEV)
        origin_b = lax.rem(my + s + 1, N_DEV)
        for j in range(N_SUB):
            rdmas[(id(comm_a), s, j)].wait_recv()
            ya = jnp.dot(comm_a[s, pl.ds(j * m_sub, m_sub), :], w_vmem[:, :],
                         preferred_element_type=jnp.float32)
            store(origin_a * m_per + j * m_sub, m_sub, ya, 5 + 2 * j)
            rdmas[(id(comm_b), s, j)].wait_recv()
            yb = jnp.dot(comm_b[s, pl.ds(j * m_sub, m_sub), :], w_vmem[:, :],
                         preferred_element_type=jnp.float32)
            store(origin_b * m_per + m_half + j * m_sub, m_sub, yb, 6 + 2 * j)

        for r in rdmas.values():
            r.wait_send()
        for cp in out_cps:
            cp.wait()

    return pl.pallas_call(
        body,
        out_shape=jax.ShapeDtypeStruct((N_DEV * m_per, n_per), jnp.float32),
        in_specs=[
            pl.BlockSpec(memory_space=pltpu.MemorySpace.HBM),
            pl.BlockSpec(memory_space=pltpu.MemorySpace.HBM),
        ],
        out_specs=pl.BlockSpec(memory_space=pltpu.MemorySpace.HBM),
        scratch_shapes=[
            pltpu.VMEM((m_per, k), jnp.float32),
            pltpu.VMEM((k, n_per), jnp.float32),
            pltpu.VMEM((N_DEV * m_per, n_per), jnp.float32),
            pltpu.VMEM((N_HOP, m_half, k), jnp.float32),
            pltpu.VMEM((N_HOP, m_half, k), jnp.float32),
            pltpu.SemaphoreType.DMA((N_HOP, N_SUB)),
            pltpu.SemaphoreType.DMA((N_HOP, N_SUB)),
            pltpu.SemaphoreType.DMA((N_HOP, N_SUB)),
            pltpu.SemaphoreType.DMA((N_HOP, N_SUB)),
            pltpu.SemaphoreType.DMA((2,)),
            pltpu.SemaphoreType.DMA((9,)),
        ],
        compiler_params=pltpu.CompilerParams(collective_id=0),
    )(x, w_mat)


# device time: 80095 ns/iter; 1.0081x vs baseline; 1.0081x over previous
import jax
import jax.numpy as jnp
from jax import lax
from jax.experimental import pallas as pl
from jax.experimental.pallas import tpu as pltpu

N_DEV = 4
N_HOP = N_DEV - 1
N_SUB = 4
_GELU_C = 0.7978845608028654


def _gelu(y):
    return 0.5 * y * (1.0 + jnp.tanh(_GELU_C * (y + 0.044715 * y * y * y)))


def kernel(x, w_mat):
    m_per, k = x.shape
    _, n_per = w_mat.shape
    m_half = m_per // 2
    m_sub = m_half // N_SUB

    def body(x_hbm, w_hbm, out_hbm,
             x_vmem, w_vmem, out_vmem, comm_a, comm_b,
             send_a, recv_a, send_b, recv_b, local_sems, out_sems):
        my = lax.axis_index("i")
        left = lax.rem(my - 1 + N_DEV, N_DEV)
        right = lax.rem(my + 1, N_DEV)

        cp_x = pltpu.make_async_copy(x_hbm, x_vmem, local_sems.at[0])
        cp_w = pltpu.make_async_copy(w_hbm, w_vmem, local_sems.at[1])
        cp_x.start()
        cp_w.start()

        barrier_sem = pltpu.get_barrier_semaphore()
        for nbr in (left, right):
            pl.semaphore_signal(
                barrier_sem, inc=1,
                device_id=(nbr,), device_id_type=pl.DeviceIdType.MESH,
            )
        pl.semaphore_wait(barrier_sem, 2)

        def make_rdma(ring_comm, sems_pair, dev, h, j):
            if h == 0:
                base = 0 if ring_comm is comm_a else m_half
                src = x_hbm.at[pl.ds(base + j * m_sub, m_sub)]
            else:
                src = ring_comm.at[h - 1, pl.ds(j * m_sub, m_sub)]
            send, recv = sems_pair
            return pltpu.make_async_remote_copy(
                src_ref=src,
                dst_ref=ring_comm.at[h, pl.ds(j * m_sub, m_sub)],
                send_sem=send.at[h, j],
                recv_sem=recv.at[h, j],
                device_id=(dev,),
                device_id_type=pl.DeviceIdType.MESH,
            )

        out_cps = []

        def store(row0, nrows, y, sem_idx):
            out_vmem[pl.ds(row0, nrows), :] = _gelu(y)
            cp = pltpu.make_async_copy(
                out_vmem.at[pl.ds(row0, nrows)],
                out_hbm.at[pl.ds(row0, nrows)],
                out_sems.at[sem_idx],
            )
            cp.start()
            out_cps.append(cp)

        rings = (
            (comm_a, (send_a, recv_a), right),
            (comm_b, (send_b, recv_b), left),
        )

        rdmas = {}
        for ring, sems, dev in rings:
            for j in range(N_SUB):
                r = make_rdma(ring, sems, dev, 0, j)
                r.start()
                rdmas[(id(ring), 0, j)] = r

        cp_w.wait()
        cp_x.wait()
        y = jnp.dot(x_vmem[:, :], w_vmem[:, :],
                    preferred_element_type=jnp.float32)
        store(my * m_per, m_per, y, 0)

        def compute(slot, sem_base):
            origin_a = lax.rem(my - slot - 1 + N_DEV, N_DEV)
            origin_b = lax.rem(my + slot + 1, N_DEV)
            ya = jnp.dot(comm_a[slot, :, :], w_vmem[:, :],
                         preferred_element_type=jnp.float32)
            store(origin_a * m_per, m_half, ya, sem_base)
            yb = jnp.dot(comm_b[slot, :, :], w_vmem[:, :],
                         preferred_element_type=jnp.float32)
            store(origin_b * m_per + m_half, m_half, yb, sem_base + 1)

        for h in range(1, N_HOP):
            for j in range(N_SUB):
                for ring, sems, dev in rings:
                    rdmas[(id(ring), h - 1, j)].wait_recv()
                    r = make_rdma(ring, sems, dev, h, j)
                    r.start()
                    rdmas[(id(ring), h, j)] = r
            compute(h - 1, 1 + 2 * (h - 1))

        s = N_HOP - 1
        origin_a = lax.rem(my - s - 1 + N_DEV, N_DEV)
        origin_b = lax.rem(my + s + 1, N_DEV)
        for j in range(N_SUB):
            rdmas[(id(comm_a), s, j)].wait_recv()
            ya = jnp.dot(comm_a[s, pl.ds(j * m_sub, m_sub), :], w_vmem[:, :],
                         preferred_element_type=jnp.float32)
            store(origin_a * m_per + j * m_sub, m_sub, ya, 5 + 2 * j)
            rdmas[(id(comm_b), s, j)].wait_recv()
            yb = jnp.dot(comm_b[s, pl.ds(j * m_sub, m_sub), :], w_vmem[:, :],
                         preferred_element_type=jnp.float32)
            store(origin_b * m_per + m_half + j * m_sub, m_sub, yb, 6 + 2 * j)

        for r in rdmas.values():
            r.wait_send()
        for cp in out_cps:
            cp.wait()

    return pl.pallas_call(
        body,
        out_shape=jax.ShapeDtypeStruct((N_DEV * m_per, n_per), jnp.float32),
        in_specs=[
            pl.BlockSpec(memory_space=pltpu.MemorySpace.HBM),
            pl.BlockSpec(memory_space=pltpu.MemorySpace.HBM),
        ],
        out_specs=pl.BlockSpec(memory_space=pltpu.MemorySpace.HBM),
        scratch_shapes=[
            pltpu.VMEM((m_per, k), jnp.float32),
            pltpu.VMEM((k, n_per), jnp.float32),
            pltpu.VMEM((N_DEV * m_per, n_per), jnp.float32),
            pltpu.VMEM((N_HOP, m_half, k), jnp.float32),
            pltpu.VMEM((N_HOP, m_half, k), jnp.float32),
            pltpu.SemaphoreType.DMA((N_HOP, N_SUB)),
            pltpu.SemaphoreType.DMA((N_HOP, N_SUB)),
            pltpu.SemaphoreType.DMA((N_HOP, N_SUB)),
            pltpu.SemaphoreType.DMA((N_HOP, N_SUB)),
            pltpu.SemaphoreType.DMA((2,)),
            pltpu.SemaphoreType.DMA((2 * N_SUB + 6,)),
        ],
        compiler_params=pltpu.CompilerParams(collective_id=0),
    )(x, w_mat)


# device time: 46516 ns/iter; 1.7358x vs baseline; 1.7219x over previous
import jax
import jax.numpy as jnp
from jax import lax
from jax.experimental import pallas as pl
from jax.experimental.pallas import tpu as pltpu

N_DEV = 4
N_HOP = N_DEV - 1
N_SUB = 2
_GELU_C = 0.7978845608028654


def _gelu(y):
    return 0.5 * y * (1.0 + jnp.tanh(_GELU_C * (y + 0.044715 * y * y * y)))


def kernel(x, w_mat):
    m_per, k = x.shape
    _, n_per = w_mat.shape
    m_half = m_per // 2
    m_sub = m_half // N_SUB

    def body(x_ref, w_ref, out_ref,
             x_bf, w_bf, comm_a, comm_b,
             send_a, recv_a, send_b, recv_b):
        my = lax.axis_index("i")
        left = lax.rem(my - 1 + N_DEV, N_DEV)
        right = lax.rem(my + 1, N_DEV)

        x_bf[:, :] = x_ref[:, :].astype(jnp.bfloat16)

        barrier_sem = pltpu.get_barrier_semaphore()
        for nbr in (left, right):
            pl.semaphore_signal(
                barrier_sem, inc=1,
                device_id=(nbr,), device_id_type=pl.DeviceIdType.MESH,
            )
        pl.semaphore_wait(barrier_sem, 2)

        def make_rdma(ring_comm, sems_pair, dev, h, j):
            if h == 0:
                base = 0 if ring_comm is comm_a else m_half
                src = x_bf.at[pl.ds(base + j * m_sub, m_sub)]
            else:
                src = ring_comm.at[h - 1, pl.ds(j * m_sub, m_sub)]
            send, recv = sems_pair
            return pltpu.make_async_remote_copy(
                src_ref=src,
                dst_ref=ring_comm.at[h, pl.ds(j * m_sub, m_sub)],
                send_sem=send.at[h, j],
                recv_sem=recv.at[h, j],
                device_id=(dev,),
                device_id_type=pl.DeviceIdType.MESH,
            )

        rings = (
            (comm_a, (send_a, recv_a), right),
            (comm_b, (send_b, recv_b), left),
        )

        rdmas = {}
        for ring, sems, dev in rings:
            for j in range(N_SUB):
                r = make_rdma(ring, sems, dev, 0, j)
                r.start()
                rdmas[(id(ring), 0, j)] = r

        w_bf[:, :] = w_ref[:, :].astype(jnp.bfloat16)
        y = jnp.dot(x_bf[:, :], w_bf[:, :],
                    preferred_element_type=jnp.float32)
        out_ref[pl.ds(my * m_per, m_per), :] = _gelu(y)

        def compute(slot):
            origin_a = lax.rem(my - slot - 1 + N_DEV, N_DEV)
            origin_b = lax.rem(my + slot + 1, N_DEV)
            ya = jnp.dot(comm_a[slot, :, :], w_bf[:, :],
                         preferred_element_type=jnp.float32)
            out_ref[pl.ds(origin_a * m_per, m_half), :] = _gelu(ya)
            yb = jnp.dot(comm_b[slot, :, :], w_bf[:, :],
                         preferred_element_type=jnp.float32)
            out_ref[pl.ds(origin_b * m_per + m_half, m_half), :] = _gelu(yb)

        for h in range(1, N_HOP):
            for j in range(N_SUB):
                for ring, sems, dev in rings:
                    rdmas[(id(ring), h - 1, j)].wait_recv()
                    r = make_rdma(ring, sems, dev, h, j)
                    r.start()
                    rdmas[(id(ring), h, j)] = r
            compute(h - 1)

        s = N_HOP - 1
        origin_a = lax.rem(my - s - 1 + N_DEV, N_DEV)
        origin_b = lax.rem(my + s + 1, N_DEV)
        for j in range(N_SUB):
            rdmas[(id(comm_a), s, j)].wait_recv()
            ya = jnp.dot(comm_a[s, pl.ds(j * m_sub, m_sub), :], w_bf[:, :],
                         preferred_element_type=jnp.float32)
            out_ref[pl.ds(origin_a * m_per + j * m_sub, m_sub), :] = _gelu(ya)
            rdmas[(id(comm_b), s, j)].wait_recv()
            yb = jnp.dot(comm_b[s, pl.ds(j * m_sub, m_sub), :], w_bf[:, :],
                         preferred_element_type=jnp.float32)
            out_ref[pl.ds(origin_b * m_per + m_half + j * m_sub, m_sub), :] = (
                _gelu(yb))

        for r in rdmas.values():
            r.wait_send()

    return pl.pallas_call(
        body,
        out_shape=jax.ShapeDtypeStruct((N_DEV * m_per, n_per), jnp.float32),
        in_specs=[
            pl.BlockSpec(memory_space=pltpu.VMEM),
            pl.BlockSpec(memory_space=pltpu.VMEM),
        ],
        out_specs=pl.BlockSpec(memory_space=pltpu.VMEM),
        scratch_shapes=[
            pltpu.VMEM((m_per, k), jnp.bfloat16),
            pltpu.VMEM((k, n_per), jnp.bfloat16),
            pltpu.VMEM((N_HOP, m_half, k), jnp.bfloat16),
            pltpu.VMEM((N_HOP, m_half, k), jnp.bfloat16),
            pltpu.SemaphoreType.DMA((N_HOP, N_SUB)),
            pltpu.SemaphoreType.DMA((N_HOP, N_SUB)),
            pltpu.SemaphoreType.DMA((N_HOP, N_SUB)),
            pltpu.SemaphoreType.DMA((N_HOP, N_SUB)),
        ],
        compiler_params=pltpu.CompilerParams(collective_id=0),
    )(x, w_mat)
